# Initial kernel scaffold; baseline (speedup 1.0000x reference)
import jax
import jax.numpy as jnp
from jax.experimental import pallas as pl


def kernel(x, edge_index, batch, Wd0, bd0, Wd1, bd1, Wd2, bd2, p1, p2, Wu0, bu0, Wu1, bu1):
    raise NotImplementedError("write your pallas kernel here")


# speedup vs baseline: 1264.5771x; 1264.5771x over previous
import functools

import jax
import jax.numpy as jnp
from jax import lax
from jax.experimental import pallas as pl
from jax.experimental.pallas import tpu as pltpu
from jax.experimental.pallas import tpu_sc as plsc

N = 10000
E = 160000
K1 = 2500
K2 = 625
H = 64
FP = 16
NP = 10240
K1P = 2560
K2P = 640
NJP = 10240

NC, NS = 2, 16
NW = NC * NS
E2 = 163840
EPW = E2 // NW
CH = 128
NCHUNK = EPW // CH

RT = 20096
ZR = RT // NS
AR = 10112
ZRB = AR // NS

RB = 10
NROUND = K1P // RB // NW
CAPT = 1600
CAP = NW * CAPT
CCH = 128
NCCH = CAP // CCH

def _sc_mesh():
    return plsc.VectorSubcoreMesh(core_axis_name="c", subcore_axis_name="s",
                                  num_cores=NC, num_subcores=NS)


def _sc_stats(srcp, dstp, zeros_zr, ones_ch):
    @functools.partial(
        pl.kernel,
        out_type=jax.ShapeDtypeStruct((2 * RT, FP), jnp.float32),
        mesh=_sc_mesh(),
        compiler_params=pltpu.CompilerParams(use_tc_tiling_on_sc=False,
                                             needs_layout_passes=False),
        scratch_types=[
            pltpu.VMEM_SHARED((RT, FP), jnp.float32),
            pltpu.VMEM((CH,), jnp.int32),
            pltpu.VMEM((CH,), jnp.int32),
            pltpu.VMEM((CH,), jnp.int32),
            pltpu.VMEM((CH, FP), jnp.float32),
        ],
    )
    def k(src_hbm, dst_hbm, z_hbm, ones_hbm, out_hbm, acc_sh, sidx_v, didx_v,
          lidx_v, ones_v):
        cid = lax.axis_index("c")
        sid = lax.axis_index("s")
        wid = cid * NS + sid
        pltpu.sync_copy(ones_hbm, ones_v)
        pltpu.sync_copy(z_hbm, acc_sh.at[pl.ds(sid * ZR, ZR)])
        plsc.subcore_barrier()
        lane = jax.lax.iota(jnp.int32, 16)
        dump = N + (lane & 7)

        def chunk(c, _):
            base = wid * EPW + c * CH
            pltpu.sync_copy(src_hbm.at[pl.ds(base, CH)], sidx_v)
            pltpu.sync_copy(dst_hbm.at[pl.ds(base, CH)], didx_v)

            def sub(i, _):
                s16 = sidx_v[pl.ds(i * 16, 16)]
                d16 = didx_v[pl.ds(i * 16, 16)]
                l16 = jnp.where(s16 == d16, s16 + (N + 8), dump)
                lidx_v[pl.ds(i * 16, 16)] = l16
                return 0

            lax.fori_loop(0, CH // 16, sub, 0)
            pltpu.sync_copy(ones_v, acc_sh.at[didx_v], add=True)
            pltpu.sync_copy(ones_v, acc_sh.at[lidx_v], add=True)
            return 0

        lax.fori_loop(0, NCHUNK, chunk, 0)
        plsc.subcore_barrier()
        pltpu.sync_copy(acc_sh.at[pl.ds(sid * ZR, ZR)],
                        out_hbm.at[pl.ds(cid * RT + sid * ZR, ZR)])

    return k(srcp, dstp, zeros_zr, ones_ch)


def _sc_aggregate(table, srcp, dstp, zeros_zrb):
    @functools.partial(
        pl.kernel,
        out_type=jax.ShapeDtypeStruct((2 * AR, FP), jnp.float32),
        mesh=_sc_mesh(),
        compiler_params=pltpu.CompilerParams(use_tc_tiling_on_sc=False,
                                             needs_layout_passes=False),
        scratch_types=[
            pltpu.VMEM_SHARED((AR, FP), jnp.float32),
            pltpu.VMEM((CH,), jnp.int32),
            pltpu.VMEM((CH,), jnp.int32),
            pltpu.VMEM((CH, FP), jnp.float32),
            pltpu.SemaphoreType.DMA,
        ],
    )
    def k(tab_hbm, src_hbm, dst_hbm, z_hbm, out_hbm, acc_sh, sidx_v, didx_v,
          rows_v, sem):
        cid = lax.axis_index("c")
        sid = lax.axis_index("s")
        wid = cid * NS + sid
        pltpu.sync_copy(z_hbm, acc_sh.at[pl.ds(sid * ZRB, ZRB)])
        plsc.subcore_barrier()

        def chunk(c, _):
            base = wid * EPW + c * CH
            pltpu.sync_copy(src_hbm.at[pl.ds(base, CH)], sidx_v)
            pltpu.sync_copy(dst_hbm.at[pl.ds(base, CH)], didx_v)
            pltpu.async_copy(tab_hbm.at[sidx_v], rows_v, sem).wait()
            pltpu.sync_copy(rows_v, acc_sh.at[didx_v], add=True)
            return 0

        lax.fori_loop(0, NCHUNK, chunk, 0)
        plsc.subcore_barrier()
        pltpu.sync_copy(acc_sh.at[pl.ds(sid * ZRB, ZRB)],
                        out_hbm.at[pl.ds(cid * AR + sid * ZRB, ZRB)])

    return k(table, srcp, dstp, zeros_zrb)


def _sc_compact(srcp, dstp, map1p, sent):
    otype = jax.ShapeDtypeStruct((CAP,), jnp.int32)

    @functools.partial(
        pl.kernel,
        out_type=(otype, otype, otype, otype),
        mesh=_sc_mesh(),
        compiler_params=pltpu.CompilerParams(needs_layout_passes=False),
        scratch_types=[
            pltpu.VMEM((AR,), jnp.int32),
            pltpu.VMEM((CH,), jnp.int32),
            pltpu.VMEM((CH,), jnp.int32),
            pltpu.VMEM((CAPT + 16,), jnp.int32),
            pltpu.VMEM((CAPT + 16,), jnp.int32),
            pltpu.VMEM((CAPT + 16,), jnp.int32),
            pltpu.VMEM((CAPT + 16,), jnp.int32),
        ],
    )
    def k(src_hbm, dst_hbm, map_hbm, sent_hbm, rl_hbm, rc_hbm, cl_hbm, cc_hbm,
          map_v, sidx_v, didx_v, brl, brc, bcl, bcc):
        cid = lax.axis_index("c")
        sid = lax.axis_index("s")
        wid = cid * NS + sid
        pltpu.sync_copy(map_hbm, map_v)
        pltpu.sync_copy(sent_hbm, brl)
        pltpu.sync_copy(sent_hbm, brc)
        pltpu.sync_copy(sent_hbm, bcl)
        pltpu.sync_copy(sent_hbm, bcc)

        def chunk(c, carry):
            cr, cc2 = carry
            base = wid * EPW + c * CH
            pltpu.sync_copy(src_hbm.at[pl.ds(base, CH)], sidx_v)
            pltpu.sync_copy(dst_hbm.at[pl.ds(base, CH)], didx_v)

            def sub(i, carry2):
                cr2, cc3 = carry2
                s16 = sidx_v[pl.ds(i * 16, 16)]
                d16 = didx_v[pl.ds(i * 16, 16)]
                mr = plsc.load_gather(map_v, [s16])
                selr = mr >= 0
                plsc.store_compressed(brl.at[pl.ds(cr2, 16)], mr, mask=selr)
                plsc.store_compressed(brc.at[pl.ds(cr2, 16)], d16, mask=selr)
                cr2 = jnp.minimum(cr2 + jnp.sum(selr.astype(jnp.int32)), CAPT)
                mc = plsc.load_gather(map_v, [d16])
                selc = mc >= 0
                plsc.store_compressed(bcl.at[pl.ds(cc3, 16)], mc, mask=selc)
                plsc.store_compressed(bcc.at[pl.ds(cc3, 16)], s16, mask=selc)
                cc3 = jnp.minimum(cc3 + jnp.sum(selc.astype(jnp.int32)), CAPT)
                return (cr2, cc3)

            return lax.fori_loop(0, CH // 16, sub, (cr, cc2))

        lax.fori_loop(0, NCHUNK, chunk, (jnp.int32(0), jnp.int32(0)))
        off = wid * CAPT
        pltpu.sync_copy(brl.at[pl.ds(0, CAPT)], rl_hbm.at[pl.ds(off, CAPT)])
        pltpu.sync_copy(brc.at[pl.ds(0, CAPT)], rc_hbm.at[pl.ds(off, CAPT)])
        pltpu.sync_copy(bcl.at[pl.ds(0, CAPT)], cl_hbm.at[pl.ds(off, CAPT)])
        pltpu.sync_copy(bcc.at[pl.ds(0, CAPT)], cc_hbm.at[pl.ds(off, CAPT)])

    return k(srcp, dstp, map1p, sent)


def _sc_build(rl, rc, cl, cc, perm1p):
    otype = jax.ShapeDtypeStruct((K1P * NP,), jnp.float32)

    @functools.partial(
        pl.kernel,
        out_type=(otype, otype),
        mesh=_sc_mesh(),
        compiler_params=pltpu.CompilerParams(needs_layout_passes=False),
        scratch_types=[
            pltpu.VMEM((RB * NP,), jnp.float32),
            pltpu.VMEM((K1P,), jnp.int32),
            pltpu.VMEM((CCH,), jnp.int32),
            pltpu.VMEM((CCH,), jnp.int32),
        ],
    )
    def k(rl_hbm, rc_hbm, cl_hbm, cc_hbm, perm_hbm, mr_hbm, mct_hbm,
          blk_v, perm_v, lr_v, co_v):
        cid = lax.axis_index("c")
        sid = lax.axis_index("s")
        wid = cid * NS + sid
        pltpu.sync_copy(perm_hbm, perm_v)
        lane = jax.lax.iota(jnp.int32, 16)
        zeros16 = jnp.zeros((16,), jnp.float32)
        ones16 = jnp.ones((16,), jnp.float32)

        for kind in range(2):
            l_hbm = rl_hbm if kind == 0 else cl_hbm
            c_hbm = rc_hbm if kind == 0 else cc_hbm
            o_hbm = mr_hbm if kind == 0 else mct_hbm
            for r in range(NROUND):
                b = r * NW + wid
                row0 = b * RB

                def zero(i, _):
                    blk_v[pl.ds(i * 16, 16)] = zeros16
                    return 0

                lax.fori_loop(0, RB * NP // 16, zero, 0)
                rows = row0 + lane
                idm = (lane < RB) & (rows < K1)
                cols = plsc.load_gather(perm_v, [jnp.minimum(rows, K1P - 1)])
                plsc.addupdate_scatter(blk_v, [lane * NP + cols], ones16,
                                       mask=idm)

                def chunk(c, _):
                    pltpu.sync_copy(l_hbm.at[pl.ds(c * CCH, CCH)], lr_v)
                    pltpu.sync_copy(c_hbm.at[pl.ds(c * CCH, CCH)], co_v)

                    def sub(i, _):
                        l16 = lr_v[pl.ds(i * 16, 16)]
                        c16 = co_v[pl.ds(i * 16, 16)]
                        m = (l16 >= row0) & (l16 < row0 + RB)
                        flat = (l16 - row0) * NP + c16
                        flat = jnp.where(m, flat, 0)
                        plsc.addupdate_scatter(blk_v, [flat], ones16, mask=m)
                        return 0

                    lax.fori_loop(0, CCH // 16, sub, 0)
                    return 0

                lax.fori_loop(0, NCCH, chunk, 0)
                pltpu.sync_copy(blk_v, o_hbm.at[pl.ds(row0 * NP, RB * NP)])

    return k(rl, rc, cl, cc, perm1p)


_PC = pl.pallas_call


def _tc_stats_finalize(stats, xpad):
    def body(st_ref, x_ref, dinv_ref, fill_ref, xws_ref):
        st = st_ref[...]
        degc = st[0:N, 0:1] + st[RT:RT + N, 0:1]
        loopc = st[N + 8:N + 8 + N, 0:1] + st[RT + N + 8:RT + N + 8 + N, 0:1]
        fill = jnp.where(loopc > 0, 0.0, 2.0)
        deg = degc + fill
        dinv = jnp.where(deg > 0, 1.0 / jnp.sqrt(jnp.maximum(deg, 1e-12)), 0.0)
        dinv_ref[...] = dinv
        fill_ref[...] = fill
        xws_ref[...] = x_ref[...] * dinv

    return _PC(
        body,
        out_shape=(
            jax.ShapeDtypeStruct((N, 1), jnp.float32),
            jax.ShapeDtypeStruct((N, 1), jnp.float32),
            jax.ShapeDtypeStruct((N, FP), jnp.float32),
        ),
    )(stats, xpad)


def _tc_conv0(parts, xpad, dinv, fill, W0p, b0, p1):
    def body(pr_ref, x_ref, dinv_ref, fill_ref, w_ref, b_ref, p_ref,
             h_ref, s_ref):
        pr = pr_ref[...]
        agg = pr[0:N, :] + pr[AR:AR + N, :]
        dinv = dinv_ref[...]
        pre = dinv * agg + (dinv * dinv * fill_ref[...]) * x_ref[...]
        h = jnp.maximum(
            jnp.dot(pre, w_ref[...], preferred_element_type=jnp.float32)
            + b_ref[...], 0.0)
        h_ref[...] = h
        p = p_ref[...]
        pn = p / jnp.sqrt(jnp.sum(p * p))
        s_ref[...] = jnp.tanh(
            jnp.dot(h, pn, preferred_element_type=jnp.float32))

    return _PC(
        body,
        out_shape=(
            jax.ShapeDtypeStruct((N, H), jnp.float32),
            jax.ShapeDtypeStruct((N, 1), jnp.float32),
        ),
    )(parts, xpad, dinv, fill, W0p, b0, p1)


def _tc_rank(s_row, s_col, nj, k, blk=256):
    njp = s_row.shape[1]
    grid = njp // blk

    def body(sr_ref, sc_ref, r_ref, m_ref):
        pid = pl.program_id(0)
        si = sr_ref[...]
        sj = sc_ref[...]
        ii = pid * blk + lax.broadcasted_iota(jnp.int32, (1, blk), 1)
        jj = lax.broadcasted_iota(jnp.int32, (nj, 1), 0)
        gt = (sj > si).astype(jnp.float32)
        eq = ((sj == si) & (jj < ii)).astype(jnp.float32)
        rank = jnp.sum(gt + eq, axis=0, keepdims=True).astype(jnp.int32)
        r_ref[...] = rank
        m_ref[...] = jnp.where(rank < k, rank, -1)

    return _PC(
        body,
        grid=(grid,),
        in_specs=[
            pl.BlockSpec((1, blk), lambda i: (0, i)),
            pl.BlockSpec((nj, 1), lambda i: (0, 0)),
        ],
        out_specs=(
            pl.BlockSpec((1, blk), lambda i: (0, i)),
            pl.BlockSpec((1, blk), lambda i: (0, i)),
        ),
        out_shape=(
            jax.ShapeDtypeStruct((1, njp), jnp.int32),
            jax.ShapeDtypeStruct((1, njp), jnp.int32),
        ),
    )(s_row, s_col)


def _tc_pool1(rank_row, h1, s1):
    blk = 128
    grid = K1P // blk

    def body(r_ref, h_ref, s_ref, xp_ref, pm_ref):
        pid = pl.program_id(0)
        rank = r_ref[...]
        aa = pid * blk + lax.broadcasted_iota(jnp.int32, (blk, 1), 0)
        oh = (rank == aa).astype(jnp.float32)
        hs = h_ref[...] * s_ref[...]
        xp_ref[...] = jnp.dot(oh, hs, preferred_element_type=jnp.float32)
        jidx = lax.broadcasted_iota(jnp.int32, (N, 1), 0).astype(jnp.float32)
        pm_ref[...] = jnp.dot(oh, jidx,
                              preferred_element_type=jnp.float32).astype(
                                  jnp.int32)

    return _PC(
        body,
        grid=(grid,),
        in_specs=[
            pl.BlockSpec((1, N), lambda i: (0, 0)),
            pl.BlockSpec((N, H), lambda i: (0, 0)),
            pl.BlockSpec((N, 1), lambda i: (0, 0)),
        ],
        out_specs=(
            pl.BlockSpec((blk, H), lambda i: (i, 0)),
            pl.BlockSpec((blk, 1), lambda i: (i, 0)),
        ),
        out_shape=(
            jax.ShapeDtypeStruct((K1P, H), jnp.float32),
            jax.ShapeDtypeStruct((K1P, 1), jnp.int32),
        ),
    )(rank_row, h1, s1)


def _tc_b1(mr, mct):
    bm, bn, bk = 256, 256, 640
    nk = NP // bk

    def body(a_ref, b_ref, o_ref):
        i, j, k = pl.program_id(0), pl.program_id(1), pl.program_id(2)

        @pl.when(k == 0)
        def _():
            o_ref[...] = jnp.zeros_like(o_ref)

        a = a_ref[...].astype(jnp.bfloat16)
        b = b_ref[...].astype(jnp.bfloat16)
        o_ref[...] += lax.dot_general(
            a, b, (((1,), (1,)), ((), ())),
            preferred_element_type=jnp.float32)

        @pl.when(k == nk - 1)
        def _():
            rows = i * bm + lax.broadcasted_iota(jnp.int32, (bm, bn), 0)
            cols = j * bn + lax.broadcasted_iota(jnp.int32, (bm, bn), 1)
            o_ref[...] = jnp.where(rows == cols, 0.0, o_ref[...])

    return _PC(
        body,
        grid=(K1P // bm, K1P // bn, nk),
        in_specs=[
            pl.BlockSpec((bm, bk), lambda i, j, k: (i, k)),
            pl.BlockSpec((bn, bk), lambda i, j, k: (j, k)),
        ],
        out_specs=pl.BlockSpec((bm, bn), lambda i, j, k: (i, j)),
        out_shape=jax.ShapeDtypeStruct((K1P, K1P), jnp.float32),
        compiler_params=pltpu.CompilerParams(
            dimension_semantics=("parallel", "parallel", "arbitrary")),
    )(mr, mct)


def _tc_gcn1_down(b1, xp, W1t, b1b, p2):
    def body(B_ref, x_ref, w_ref, bb_ref, p_ref, h_ref, s_ref, d_ref):
        B = B_ref[...]
        valid = lax.broadcasted_iota(jnp.int32, (K1P, 1), 0) < K1
        deg = jnp.sum(B, axis=0)[:, None] + 2.0
        dinv = 1.0 / jnp.sqrt(deg)
        d_ref[...] = dinv
        xw = jnp.dot(x_ref[...], w_ref[...],
                     preferred_element_type=jnp.float32)
        agg = lax.dot_general(B, dinv * xw, (((0,), (0,)), ((), ())),
                              preferred_element_type=jnp.float32)
        h = jnp.maximum(dinv * agg + 2.0 * dinv * dinv * xw + bb_ref[...], 0.0)
        h = jnp.where(valid, h, 0.0)
        h_ref[...] = h
        p = p_ref[...]
        pn = p / jnp.sqrt(jnp.sum(p * p))
        s = jnp.tanh(jnp.dot(h, pn, preferred_element_type=jnp.float32))
        s_ref[...] = jnp.where(valid, s, -2.0)

    return _PC(
        body,
        out_shape=(
            jax.ShapeDtypeStruct((K1P, H), jnp.float32),
            jax.ShapeDtypeStruct((K1P, 1), jnp.float32),
            jax.ShapeDtypeStruct((K1P, 1), jnp.float32),
        ),
    )(b1, xp, W1t, b1b, p2)


def _tc_level2(rank2_row, h2, s2, b1):
    def body(r_ref, h_ref, s_ref, B_ref, xp_ref, b2_ref):
        rank = r_ref[...]
        aa = lax.broadcasted_iota(jnp.int32, (K2P, 1), 0)
        oh = (rank == aa).astype(jnp.float32)
        xp_ref[...] = jnp.dot(oh, h_ref[...] * s_ref[...],
                              preferred_element_type=jnp.float32)
        B = B_ref[...]
        rg = jnp.dot(oh, B, preferred_element_type=jnp.float32) + oh
        cg = lax.dot_general(rg, oh, (((1,), (1,)), ((), ())),
                             preferred_element_type=jnp.float32)
        bo = lax.dot_general(B, oh, (((1,), (1,)), ((), ())),
                             preferred_element_type=jnp.float32)
        b2 = jnp.dot(rg, bo, preferred_element_type=jnp.float32) + cg
        rows = lax.broadcasted_iota(jnp.int32, (K2P, K2P), 0)
        cols = lax.broadcasted_iota(jnp.int32, (K2P, K2P), 1)
        b2_ref[...] = jnp.where(rows == cols, 0.0, b2)

    return _PC(
        body,
        out_shape=(
            jax.ShapeDtypeStruct((K2P, H), jnp.float32),
            jax.ShapeDtypeStruct((K2P, K2P), jnp.float32),
        ),
    )(rank2_row, h2, s2, b1)


def _tc_bottom_up(rank2_row, xp2, b2, h2, b1, dinv1, W2t, b2b, Wu0t, bu0):
    def body(r_ref, x2_ref, B2_ref, h2_ref, B1_ref, d1_ref, w2_ref, bb2_ref,
             wu_ref, bbu_ref, hu_ref):
        B2 = B2_ref[...]
        valid2 = lax.broadcasted_iota(jnp.int32, (K2P, 1), 0) < K2
        deg2 = jnp.sum(B2, axis=0)[:, None] + 2.0
        dinv2 = 1.0 / jnp.sqrt(deg2)
        xw2 = jnp.dot(x2_ref[...], w2_ref[...],
                      preferred_element_type=jnp.float32)
        agg2 = lax.dot_general(B2, dinv2 * xw2, (((0,), (0,)), ((), ())),
                               preferred_element_type=jnp.float32)
        h3 = jnp.maximum(
            dinv2 * agg2 + 2.0 * dinv2 * dinv2 * xw2 + bb2_ref[...], 0.0)
        h3 = jnp.where(valid2, h3, 0.0)
        rank = r_ref[...]
        aa = lax.broadcasted_iota(jnp.int32, (K2P, 1), 0)
        oh = (rank == aa).astype(jnp.float32)
        up = lax.dot_general(oh, h3, (((0,), (0,)), ((), ())),
                             preferred_element_type=jnp.float32)
        xu = h2_ref[...] + up
        xwu = jnp.dot(xu, wu_ref[...], preferred_element_type=jnp.float32)
        B1 = B1_ref[...]
        d1 = d1_ref[...]
        aggu = lax.dot_general(B1, d1 * xwu, (((0,), (0,)), ((), ())),
                               preferred_element_type=jnp.float32)
        hu = jnp.maximum(d1 * aggu + 2.0 * d1 * d1 * xwu + bbu_ref[...], 0.0)
        valid1 = lax.broadcasted_iota(jnp.int32, (K1P, 1), 0) < K1
        hu_ref[...] = jnp.where(valid1, hu, 0.0)

    return _PC(
        body,
        out_shape=jax.ShapeDtypeStruct((K1P, H), jnp.float32),
    )(rank2_row, xp2, b2, h2, b1, dinv1, W2t, b2b, Wu0t, bu0)


def _tc_unpool0(rank1_col, h1, dinv, hu, Wu1p):
    blk = 200
    grid = N // blk

    def body(r_ref, h_ref, d_ref, hu_ref, w_ref, o_ref):
        rank = r_ref[...]
        aa = lax.broadcasted_iota(jnp.int32, (1, K1P), 1)
        oh = (rank == aa).astype(jnp.float32)
        up = jnp.dot(oh, hu_ref[...], preferred_element_type=jnp.float32)
        xf = h_ref[...] + up
        xwf = jnp.dot(xf, w_ref[...], preferred_element_type=jnp.float32)
        o_ref[...] = d_ref[...] * xwf

    return _PC(
        body,
        grid=(grid,),
        in_specs=[
            pl.BlockSpec((blk, 1), lambda i: (i, 0)),
            pl.BlockSpec((blk, H), lambda i: (i, 0)),
            pl.BlockSpec((blk, 1), lambda i: (i, 0)),
            pl.BlockSpec((K1P, H), lambda i: (0, 0)),
            pl.BlockSpec((H, FP), lambda i: (0, 0)),
        ],
        out_specs=pl.BlockSpec((blk, FP), lambda i: (i, 0)),
        out_shape=jax.ShapeDtypeStruct((N, FP), jnp.float32),
    )(rank1_col, h1, dinv, hu, Wu1p)


def _tc_final(parts, xwfs, dinv, fill, bu1):
    def body(pr_ref, t_ref, d_ref, f_ref, b_ref, o_ref):
        pr = pr_ref[...]
        agg = pr[0:N, :] + pr[AR:AR + N, :]
        res = d_ref[...] * (agg + f_ref[...] * t_ref[...])
        o_ref[...] = res[:, 0:7] + b_ref[...]

    return _PC(
        body,
        out_shape=jax.ShapeDtypeStruct((N, 7), jnp.float32),
    )(parts, xwfs, dinv, fill, bu1)


def kernel(x, edge_index, batch, Wd0, bd0, Wd1, bd1, Wd2, bd2, p1, p2,
           Wu0, bu0, Wu1, bu1):
    f32 = jnp.float32
    src = edge_index[0].astype(jnp.int32)
    dst = edge_index[1].astype(jnp.int32)

    npad = E2 - E
    kmod = jnp.arange(npad, dtype=jnp.int32) & 7
    srcp = jnp.concatenate([src, N + 8 + kmod])
    dstp = jnp.concatenate([dst, N + kmod])
    zeros_zr = jnp.zeros((ZR, FP), f32)
    zeros_zrb = jnp.zeros((ZRB, FP), f32)
    ones_ch = jnp.ones((CH, FP), f32)
    sent = jnp.full((CAPT + 16,), -1, jnp.int32)
    xpad = jnp.concatenate([x, jnp.zeros((N, FP - 7), f32)], axis=1)

    stats = _sc_stats(srcp, dstp, zeros_zr, ones_ch)
    dinv, fill, xws0 = _tc_stats_finalize(stats, xpad)
    table0 = jnp.concatenate([xws0, jnp.zeros((AR - N, FP), f32)], axis=0)
    parts0 = _sc_aggregate(table0, srcp, dstp, zeros_zrb)
    W0p = jnp.concatenate([Wd0.T, jnp.zeros((FP - 7, H), f32)], axis=0)
    h1, s1 = _tc_conv0(parts0, xpad, dinv, fill, W0p, bd0[None, :],
                       p1[:, None])

    s1_row = jnp.concatenate([s1[:, 0], jnp.full((NJP - N,), -2.0, f32)])
    rank1_row, map1_row = _tc_rank(s1_row[None, :], s1, N, K1)
    rank1 = rank1_row[:, :N]
    map1p = jnp.concatenate(
        [map1_row[0, :N], jnp.full((AR - N,), -1, jnp.int32)])
    xp, perm1 = _tc_pool1(rank1, h1, s1)

    rl, rc, cl, cc = _sc_compact(srcp, dstp, map1p, sent)
    mr, mct = _sc_build(rl, rc, cl, cc, perm1[:, 0])
    b1 = _tc_b1(mr.reshape(K1P, NP), mct.reshape(K1P, NP))

    h2, s2, dinv1 = _tc_gcn1_down(b1, xp, Wd1.T, bd1[None, :], p2[:, None])
    rank2_row, _ = _tc_rank(s2[:, 0][None, :], s2, K1P, K2)
    xp2, b2 = _tc_level2(rank2_row, h2, s2, b1)
    hu = _tc_bottom_up(rank2_row, xp2, b2, h2, b1, dinv1, Wd2.T,
                       bd2[None, :], Wu0.T, bu0[None, :])

    Wu1p = jnp.concatenate([Wu1.T, jnp.zeros((H, FP - 7), f32)], axis=1)
    xwfs = _tc_unpool0(rank1.reshape(N, 1), h1, dinv, hu, Wu1p)
    tablef = jnp.concatenate([xwfs, jnp.zeros((AR - N, FP), f32)], axis=0)
    partsf = _sc_aggregate(tablef, srcp, dstp, zeros_zrb)
    out = _tc_final(partsf, xwfs, dinv, fill, bu1[None, :])
    return out


# speedup vs baseline: 2331.1028x; 1.8434x over previous
import functools

import jax
import jax.numpy as jnp
from jax import lax
from jax.experimental import pallas as pl
from jax.experimental.pallas import tpu as pltpu
from jax.experimental.pallas import tpu_sc as plsc

N = 10000
E = 160000
K1 = 2500
K2 = 625
H = 64
FP = 16
NP = 10240
K1P = 2560
K2P = 640
NJP = 10240

NC, NS = 2, 16
NW = NC * NS
E2 = 163840
EPW = E2 // NW
CH = 128
NCHUNK = EPW // CH

RT = 20096
ZR = RT // NS
AR = 10112
ZRB = AR // NS

RB = 10
NROUND = K1P // RB // NW
CAPT = 1600
CAP = NW * CAPT
CCH = 512
NCCH = CAP // CCH

def _sc_mesh():
    return plsc.VectorSubcoreMesh(core_axis_name="c", subcore_axis_name="s",
                                  num_cores=NC, num_subcores=NS)


def _sc_stats(srcp, dstp, zeros_zr, ones_ch):
    @functools.partial(
        pl.kernel,
        out_type=jax.ShapeDtypeStruct((2 * RT, FP), jnp.float32),
        mesh=_sc_mesh(),
        compiler_params=pltpu.CompilerParams(use_tc_tiling_on_sc=False,
                                             needs_layout_passes=False),
        scratch_types=[
            pltpu.VMEM_SHARED((RT, FP), jnp.float32),
            pltpu.VMEM((CH,), jnp.int32),
            pltpu.VMEM((CH,), jnp.int32),
            pltpu.VMEM((CH,), jnp.int32),
            pltpu.VMEM((CH, FP), jnp.float32),
        ],
    )
    def k(src_hbm, dst_hbm, z_hbm, ones_hbm, out_hbm, acc_sh, sidx_v, didx_v,
          lidx_v, ones_v):
        cid = lax.axis_index("c")
        sid = lax.axis_index("s")
        wid = cid * NS + sid
        pltpu.sync_copy(ones_hbm, ones_v)
        pltpu.sync_copy(z_hbm, acc_sh.at[pl.ds(sid * ZR, ZR)])
        plsc.subcore_barrier()
        lane = jax.lax.iota(jnp.int32, 16)
        dump = N + (lane & 7)

        def chunk(c, _):
            base = wid * EPW + c * CH
            pltpu.sync_copy(src_hbm.at[pl.ds(base, CH)], sidx_v)
            pltpu.sync_copy(dst_hbm.at[pl.ds(base, CH)], didx_v)

            def sub(i, _):
                s16 = sidx_v[pl.ds(i * 16, 16)]
                d16 = didx_v[pl.ds(i * 16, 16)]
                l16 = jnp.where(s16 == d16, s16 + (N + 8), dump)
                lidx_v[pl.ds(i * 16, 16)] = l16
                return 0

            lax.fori_loop(0, CH // 16, sub, 0)
            pltpu.sync_copy(ones_v, acc_sh.at[didx_v], add=True)
            pltpu.sync_copy(ones_v, acc_sh.at[lidx_v], add=True)
            return 0

        lax.fori_loop(0, NCHUNK, chunk, 0)
        plsc.subcore_barrier()
        pltpu.sync_copy(acc_sh.at[pl.ds(sid * ZR, ZR)],
                        out_hbm.at[pl.ds(cid * RT + sid * ZR, ZR)])

    return k(srcp, dstp, zeros_zr, ones_ch)


def _sc_aggregate(table, srcp, dstp, zeros_zrb):
    @functools.partial(
        pl.kernel,
        out_type=jax.ShapeDtypeStruct((2 * AR, FP), jnp.float32),
        mesh=_sc_mesh(),
        compiler_params=pltpu.CompilerParams(use_tc_tiling_on_sc=False,
                                             needs_layout_passes=False),
        scratch_types=[
            pltpu.VMEM_SHARED((AR, FP), jnp.float32),
            pltpu.VMEM((CH,), jnp.int32),
            pltpu.VMEM((CH,), jnp.int32),
            pltpu.VMEM((CH, FP), jnp.float32),
            pltpu.SemaphoreType.DMA,
        ],
    )
    def k(tab_hbm, src_hbm, dst_hbm, z_hbm, out_hbm, acc_sh, sidx_v, didx_v,
          rows_v, sem):
        cid = lax.axis_index("c")
        sid = lax.axis_index("s")
        wid = cid * NS + sid
        pltpu.sync_copy(z_hbm, acc_sh.at[pl.ds(sid * ZRB, ZRB)])
        plsc.subcore_barrier()

        def chunk(c, _):
            base = wid * EPW + c * CH
            pltpu.sync_copy(src_hbm.at[pl.ds(base, CH)], sidx_v)
            pltpu.sync_copy(dst_hbm.at[pl.ds(base, CH)], didx_v)
            pltpu.async_copy(tab_hbm.at[sidx_v], rows_v, sem).wait()
            pltpu.sync_copy(rows_v, acc_sh.at[didx_v], add=True)
            return 0

        lax.fori_loop(0, NCHUNK, chunk, 0)
        plsc.subcore_barrier()
        pltpu.sync_copy(acc_sh.at[pl.ds(sid * ZRB, ZRB)],
                        out_hbm.at[pl.ds(cid * AR + sid * ZRB, ZRB)])

    return k(table, srcp, dstp, zeros_zrb)


def _sc_compact(srcp, dstp, map1p, sent):
    otype = jax.ShapeDtypeStruct((CAP,), jnp.int32)

    @functools.partial(
        pl.kernel,
        out_type=(otype, otype, otype, otype),
        mesh=_sc_mesh(),
        compiler_params=pltpu.CompilerParams(needs_layout_passes=False),
        scratch_types=[
            pltpu.VMEM((AR,), jnp.int32),
            pltpu.VMEM((CH,), jnp.int32),
            pltpu.VMEM((CH,), jnp.int32),
            pltpu.VMEM((CAPT + 16,), jnp.int32),
            pltpu.VMEM((CAPT + 16,), jnp.int32),
            pltpu.VMEM((CAPT + 16,), jnp.int32),
            pltpu.VMEM((CAPT + 16,), jnp.int32),
        ],
    )
    def k(src_hbm, dst_hbm, map_hbm, sent_hbm, rl_hbm, rc_hbm, cl_hbm, cc_hbm,
          map_v, sidx_v, didx_v, brl, brc, bcl, bcc):
        cid = lax.axis_index("c")
        sid = lax.axis_index("s")
        wid = cid * NS + sid
        pltpu.sync_copy(map_hbm, map_v)
        pltpu.sync_copy(sent_hbm, brl)
        pltpu.sync_copy(sent_hbm, brc)
        pltpu.sync_copy(sent_hbm, bcl)
        pltpu.sync_copy(sent_hbm, bcc)

        def chunk(c, carry):
            cr, cc2 = carry
            base = wid * EPW + c * CH
            pltpu.sync_copy(src_hbm.at[pl.ds(base, CH)], sidx_v)
            pltpu.sync_copy(dst_hbm.at[pl.ds(base, CH)], didx_v)

            def sub(i, carry2):
                cr2, cc3 = carry2
                s16 = sidx_v[pl.ds(i * 16, 16)]
                d16 = didx_v[pl.ds(i * 16, 16)]
                mr = plsc.load_gather(map_v, [s16])
                selr = mr >= 0
                plsc.store_compressed(brl.at[pl.ds(cr2, 16)], mr, mask=selr)
                plsc.store_compressed(brc.at[pl.ds(cr2, 16)], d16, mask=selr)
                cr2 = jnp.minimum(cr2 + jnp.sum(selr.astype(jnp.int32)), CAPT)
                mc = plsc.load_gather(map_v, [d16])
                selc = mc >= 0
                plsc.store_compressed(bcl.at[pl.ds(cc3, 16)], mc, mask=selc)
                plsc.store_compressed(bcc.at[pl.ds(cc3, 16)], s16, mask=selc)
                cc3 = jnp.minimum(cc3 + jnp.sum(selc.astype(jnp.int32)), CAPT)
                return (cr2, cc3)

            return lax.fori_loop(0, CH // 16, sub, (cr, cc2))

        lax.fori_loop(0, NCHUNK, chunk, (jnp.int32(0), jnp.int32(0)))
        off = wid * CAPT
        pltpu.sync_copy(brl.at[pl.ds(0, CAPT)], rl_hbm.at[pl.ds(off, CAPT)])
        pltpu.sync_copy(brc.at[pl.ds(0, CAPT)], rc_hbm.at[pl.ds(off, CAPT)])
        pltpu.sync_copy(bcl.at[pl.ds(0, CAPT)], cl_hbm.at[pl.ds(off, CAPT)])
        pltpu.sync_copy(bcc.at[pl.ds(0, CAPT)], cc_hbm.at[pl.ds(off, CAPT)])

    return k(srcp, dstp, map1p, sent)


def _sc_build(rl, rc, cl, cc, perm1p):
    otype = jax.ShapeDtypeStruct((K1P * NP,), jnp.float32)

    @functools.partial(
        pl.kernel,
        out_type=(otype, otype),
        mesh=_sc_mesh(),
        compiler_params=pltpu.CompilerParams(needs_layout_passes=False),
        scratch_types=[
            pltpu.VMEM((RB * NP,), jnp.float32),
            pltpu.VMEM((K1P,), jnp.int32),
            pltpu.VMEM((CCH,), jnp.int32),
            pltpu.VMEM((CCH,), jnp.int32),
        ],
    )
    def k(rl_hbm, rc_hbm, cl_hbm, cc_hbm, perm_hbm, mr_hbm, mct_hbm,
          blk_v, perm_v, lr_v, co_v):
        cid = lax.axis_index("c")
        sid = lax.axis_index("s")
        wid = cid * NS + sid
        pltpu.sync_copy(perm_hbm, perm_v)
        lane = jax.lax.iota(jnp.int32, 16)
        zeros16 = jnp.zeros((16,), jnp.float32)
        ones16 = jnp.ones((16,), jnp.float32)

        for kind in range(2):
            l_hbm = rl_hbm if kind == 0 else cl_hbm
            c_hbm = rc_hbm if kind == 0 else cc_hbm
            o_hbm = mr_hbm if kind == 0 else mct_hbm
            for r in range(NROUND):
                b = r * NW + wid
                row0 = b * RB

                def zero(i, _):
                    blk_v[pl.ds(i * 16, 16)] = zeros16
                    return 0

                lax.fori_loop(0, RB * NP // 16, zero, 0)
                rows = row0 + lane
                idm = (lane < RB) & (rows < K1)
                cols = plsc.load_gather(perm_v, [jnp.minimum(rows, K1P - 1)])
                plsc.addupdate_scatter(blk_v, [lane * NP + cols], ones16,
                                       mask=idm)

                def chunk(c, _):
                    pltpu.sync_copy(l_hbm.at[pl.ds(c * CCH, CCH)], lr_v)
                    pltpu.sync_copy(c_hbm.at[pl.ds(c * CCH, CCH)], co_v)

                    def sub(i, _):
                        l16 = lr_v[pl.ds(i * 16, 16)]
                        c16 = co_v[pl.ds(i * 16, 16)]
                        m = (l16 >= row0) & (l16 < row0 + RB)
                        flat = (l16 - row0) * NP + c16
                        flat = jnp.where(m, flat, 0)
                        plsc.addupdate_scatter(blk_v, [flat], ones16, mask=m)
                        return 0

                    lax.fori_loop(0, CCH // 16, sub, 0)
                    return 0

                lax.fori_loop(0, NCCH, chunk, 0)
                pltpu.sync_copy(blk_v, o_hbm.at[pl.ds(row0 * NP, RB * NP)])

    return k(rl, rc, cl, cc, perm1p)


_PC = pl.pallas_call


def _tc_stats_finalize(stats, xpad):
    def body(st_ref, x_ref, dinv_ref, fill_ref, xws_ref):
        st = st_ref[...]
        degc = st[0:N, 0:1] + st[RT:RT + N, 0:1]
        loopc = st[N + 8:N + 8 + N, 0:1] + st[RT + N + 8:RT + N + 8 + N, 0:1]
        fill = jnp.where(loopc > 0, 0.0, 2.0)
        deg = degc + fill
        dinv = jnp.where(deg > 0, 1.0 / jnp.sqrt(jnp.maximum(deg, 1e-12)), 0.0)
        dinv_ref[...] = dinv
        fill_ref[...] = fill
        xws_ref[...] = x_ref[...] * dinv

    return _PC(
        body,
        out_shape=(
            jax.ShapeDtypeStruct((N, 1), jnp.float32),
            jax.ShapeDtypeStruct((N, 1), jnp.float32),
            jax.ShapeDtypeStruct((N, FP), jnp.float32),
        ),
    )(stats, xpad)


def _tc_conv0(parts, xpad, dinv, fill, W0p, b0, p1):
    def body(pr_ref, x_ref, dinv_ref, fill_ref, w_ref, b_ref, p_ref,
             h_ref, s_ref):
        pr = pr_ref[...]
        agg = pr[0:N, :] + pr[AR:AR + N, :]
        dinv = dinv_ref[...]
        pre = dinv * agg + (dinv * dinv * fill_ref[...]) * x_ref[...]
        h = jnp.maximum(
            jnp.dot(pre, w_ref[...], preferred_element_type=jnp.float32)
            + b_ref[...], 0.0)
        h_ref[...] = h
        p = p_ref[...]
        pn = p / jnp.sqrt(jnp.sum(p * p))
        s_ref[...] = jnp.tanh(
            jnp.dot(h, pn, preferred_element_type=jnp.float32))

    return _PC(
        body,
        out_shape=(
            jax.ShapeDtypeStruct((N, H), jnp.float32),
            jax.ShapeDtypeStruct((N, 1), jnp.float32),
        ),
    )(parts, xpad, dinv, fill, W0p, b0, p1)


def _tc_rank(s_row, s_col, nj, k, blk=256):
    njp = s_row.shape[1]
    grid = njp // blk

    def body(sr_ref, sc_ref, r_ref, m_ref):
        pid = pl.program_id(0)
        si = sr_ref[...]
        sj = sc_ref[...]
        ii = pid * blk + lax.broadcasted_iota(jnp.int32, (1, blk), 1)
        jj = lax.broadcasted_iota(jnp.int32, (nj, 1), 0)
        gt = (sj > si).astype(jnp.float32)
        eq = ((sj == si) & (jj < ii)).astype(jnp.float32)
        rank = jnp.sum(gt + eq, axis=0, keepdims=True).astype(jnp.int32)
        r_ref[...] = rank
        m_ref[...] = jnp.where(rank < k, rank, -1)

    return _PC(
        body,
        grid=(grid,),
        in_specs=[
            pl.BlockSpec((1, blk), lambda i: (0, i)),
            pl.BlockSpec((nj, 1), lambda i: (0, 0)),
        ],
        out_specs=(
            pl.BlockSpec((1, blk), lambda i: (0, i)),
            pl.BlockSpec((1, blk), lambda i: (0, i)),
        ),
        out_shape=(
            jax.ShapeDtypeStruct((1, njp), jnp.int32),
            jax.ShapeDtypeStruct((1, njp), jnp.int32),
        ),
    )(s_row, s_col)


def _tc_pool1(rank_row, h1, s1):
    blk = 128
    grid = K1P // blk

    def body(r_ref, h_ref, s_ref, xp_ref, pm_ref):
        pid = pl.program_id(0)
        rank = r_ref[...]
        aa = pid * blk + lax.broadcasted_iota(jnp.int32, (blk, 1), 0)
        oh = (rank == aa).astype(jnp.float32)
        hs = h_ref[...] * s_ref[...]
        xp_ref[...] = jnp.dot(oh, hs, preferred_element_type=jnp.float32)
        jidx = lax.broadcasted_iota(jnp.int32, (N, 1), 0).astype(jnp.float32)
        pm_ref[...] = jnp.dot(oh, jidx,
                              preferred_element_type=jnp.float32).astype(
                                  jnp.int32)

    return _PC(
        body,
        grid=(grid,),
        in_specs=[
            pl.BlockSpec((1, N), lambda i: (0, 0)),
            pl.BlockSpec((N, H), lambda i: (0, 0)),
            pl.BlockSpec((N, 1), lambda i: (0, 0)),
        ],
        out_specs=(
            pl.BlockSpec((blk, H), lambda i: (i, 0)),
            pl.BlockSpec((blk, 1), lambda i: (i, 0)),
        ),
        out_shape=(
            jax.ShapeDtypeStruct((K1P, H), jnp.float32),
            jax.ShapeDtypeStruct((K1P, 1), jnp.int32),
        ),
    )(rank_row, h1, s1)


def _tc_b1(mr, mct):
    bm, bn, bk = 256, 256, 640
    nk = NP // bk

    def body(a_ref, b_ref, o_ref):
        i, j, k = pl.program_id(0), pl.program_id(1), pl.program_id(2)

        @pl.when(k == 0)
        def _():
            o_ref[...] = jnp.zeros_like(o_ref)

        a = a_ref[...].astype(jnp.bfloat16)
        b = b_ref[...].astype(jnp.bfloat16)
        o_ref[...] += lax.dot_general(
            a, b, (((1,), (1,)), ((), ())),
            preferred_element_type=jnp.float32)

        @pl.when(k == nk - 1)
        def _():
            rows = i * bm + lax.broadcasted_iota(jnp.int32, (bm, bn), 0)
            cols = j * bn + lax.broadcasted_iota(jnp.int32, (bm, bn), 1)
            o_ref[...] = jnp.where(rows == cols, 0.0, o_ref[...])

    return _PC(
        body,
        grid=(K1P // bm, K1P // bn, nk),
        in_specs=[
            pl.BlockSpec((bm, bk), lambda i, j, k: (i, k)),
            pl.BlockSpec((bn, bk), lambda i, j, k: (j, k)),
        ],
        out_specs=pl.BlockSpec((bm, bn), lambda i, j, k: (i, j)),
        out_shape=jax.ShapeDtypeStruct((K1P, K1P), jnp.float32),
        compiler_params=pltpu.CompilerParams(
            dimension_semantics=("parallel", "parallel", "arbitrary")),
    )(mr, mct)


def _tc_gcn1_down(b1, xp, W1t, b1b, p2):
    def body(B_ref, x_ref, w_ref, bb_ref, p_ref, h_ref, s_ref, d_ref):
        B = B_ref[...]
        valid = lax.broadcasted_iota(jnp.int32, (K1P, 1), 0) < K1
        deg = jnp.sum(B, axis=0)[:, None] + 2.0
        dinv = 1.0 / jnp.sqrt(deg)
        d_ref[...] = dinv
        xw = jnp.dot(x_ref[...], w_ref[...],
                     preferred_element_type=jnp.float32)
        agg = lax.dot_general(B, dinv * xw, (((0,), (0,)), ((), ())),
                              preferred_element_type=jnp.float32)
        h = jnp.maximum(dinv * agg + 2.0 * dinv * dinv * xw + bb_ref[...], 0.0)
        h = jnp.where(valid, h, 0.0)
        h_ref[...] = h
        p = p_ref[...]
        pn = p / jnp.sqrt(jnp.sum(p * p))
        s = jnp.tanh(jnp.dot(h, pn, preferred_element_type=jnp.float32))
        s_ref[...] = jnp.where(valid, s, -2.0)

    return _PC(
        body,
        out_shape=(
            jax.ShapeDtypeStruct((K1P, H), jnp.float32),
            jax.ShapeDtypeStruct((K1P, 1), jnp.float32),
            jax.ShapeDtypeStruct((K1P, 1), jnp.float32),
        ),
    )(b1, xp, W1t, b1b, p2)


def _tc_level2(rank2_row, h2, s2, b1):
    def body(r_ref, h_ref, s_ref, B_ref, xp_ref, b2_ref):
        rank = r_ref[...]
        aa = lax.broadcasted_iota(jnp.int32, (K2P, 1), 0)
        oh = (rank == aa).astype(jnp.float32)
        xp_ref[...] = jnp.dot(oh, h_ref[...] * s_ref[...],
                              preferred_element_type=jnp.float32)
        B = B_ref[...]
        rg = jnp.dot(oh, B, preferred_element_type=jnp.float32) + oh
        cg = lax.dot_general(rg, oh, (((1,), (1,)), ((), ())),
                             preferred_element_type=jnp.float32)
        bo = lax.dot_general(B, oh, (((1,), (1,)), ((), ())),
                             preferred_element_type=jnp.float32)
        b2 = jnp.dot(rg, bo, preferred_element_type=jnp.float32) + cg
        rows = lax.broadcasted_iota(jnp.int32, (K2P, K2P), 0)
        cols = lax.broadcasted_iota(jnp.int32, (K2P, K2P), 1)
        b2_ref[...] = jnp.where(rows == cols, 0.0, b2)

    return _PC(
        body,
        out_shape=(
            jax.ShapeDtypeStruct((K2P, H), jnp.float32),
            jax.ShapeDtypeStruct((K2P, K2P), jnp.float32),
        ),
    )(rank2_row, h2, s2, b1)


def _tc_bottom_up(rank2_row, xp2, b2, h2, b1, dinv1, W2t, b2b, Wu0t, bu0):
    def body(r_ref, x2_ref, B2_ref, h2_ref, B1_ref, d1_ref, w2_ref, bb2_ref,
             wu_ref, bbu_ref, hu_ref):
        B2 = B2_ref[...]
        valid2 = lax.broadcasted_iota(jnp.int32, (K2P, 1), 0) < K2
        deg2 = jnp.sum(B2, axis=0)[:, None] + 2.0
        dinv2 = 1.0 / jnp.sqrt(deg2)
        xw2 = jnp.dot(x2_ref[...], w2_ref[...],
                      preferred_element_type=jnp.float32)
        agg2 = lax.dot_general(B2, dinv2 * xw2, (((0,), (0,)), ((), ())),
                               preferred_element_type=jnp.float32)
        h3 = jnp.maximum(
            dinv2 * agg2 + 2.0 * dinv2 * dinv2 * xw2 + bb2_ref[...], 0.0)
        h3 = jnp.where(valid2, h3, 0.0)
        rank = r_ref[...]
        aa = lax.broadcasted_iota(jnp.int32, (K2P, 1), 0)
        oh = (rank == aa).astype(jnp.float32)
        up = lax.dot_general(oh, h3, (((0,), (0,)), ((), ())),
                             preferred_element_type=jnp.float32)
        xu = h2_ref[...] + up
        xwu = jnp.dot(xu, wu_ref[...], preferred_element_type=jnp.float32)
        B1 = B1_ref[...]
        d1 = d1_ref[...]
        aggu = lax.dot_general(B1, d1 * xwu, (((0,), (0,)), ((), ())),
                               preferred_element_type=jnp.float32)
        hu = jnp.maximum(d1 * aggu + 2.0 * d1 * d1 * xwu + bbu_ref[...], 0.0)
        valid1 = lax.broadcasted_iota(jnp.int32, (K1P, 1), 0) < K1
        hu_ref[...] = jnp.where(valid1, hu, 0.0)

    return _PC(
        body,
        out_shape=jax.ShapeDtypeStruct((K1P, H), jnp.float32),
    )(rank2_row, xp2, b2, h2, b1, dinv1, W2t, b2b, Wu0t, bu0)


def _tc_unpool0(rank1_col, h1, dinv, hu, Wu1p):
    blk = 200
    grid = N // blk

    def body(r_ref, h_ref, d_ref, hu_ref, w_ref, o_ref):
        rank = r_ref[...]
        aa = lax.broadcasted_iota(jnp.int32, (1, K1P), 1)
        oh = (rank == aa).astype(jnp.float32)
        up = jnp.dot(oh, hu_ref[...], preferred_element_type=jnp.float32)
        xf = h_ref[...] + up
        xwf = jnp.dot(xf, w_ref[...], preferred_element_type=jnp.float32)
        o_ref[...] = d_ref[...] * xwf

    return _PC(
        body,
        grid=(grid,),
        in_specs=[
            pl.BlockSpec((blk, 1), lambda i: (i, 0)),
            pl.BlockSpec((blk, H), lambda i: (i, 0)),
            pl.BlockSpec((blk, 1), lambda i: (i, 0)),
            pl.BlockSpec((K1P, H), lambda i: (0, 0)),
            pl.BlockSpec((H, FP), lambda i: (0, 0)),
        ],
        out_specs=pl.BlockSpec((blk, FP), lambda i: (i, 0)),
        out_shape=jax.ShapeDtypeStruct((N, FP), jnp.float32),
    )(rank1_col, h1, dinv, hu, Wu1p)


def _tc_final(parts, xwfs, dinv, fill, bu1):
    def body(pr_ref, t_ref, d_ref, f_ref, b_ref, o_ref):
        pr = pr_ref[...]
        agg = pr[0:N, :] + pr[AR:AR + N, :]
        res = d_ref[...] * (agg + f_ref[...] * t_ref[...])
        o_ref[...] = res[:, 0:7] + b_ref[...]

    return _PC(
        body,
        out_shape=jax.ShapeDtypeStruct((N, 7), jnp.float32),
    )(parts, xwfs, dinv, fill, bu1)


def kernel(x, edge_index, batch, Wd0, bd0, Wd1, bd1, Wd2, bd2, p1, p2,
           Wu0, bu0, Wu1, bu1):
    f32 = jnp.float32
    src = edge_index[0].astype(jnp.int32)
    dst = edge_index[1].astype(jnp.int32)

    npad = E2 - E
    kmod = jnp.arange(npad, dtype=jnp.int32) & 7
    srcp = jnp.concatenate([src, N + 8 + kmod])
    dstp = jnp.concatenate([dst, N + kmod])
    zeros_zr = jnp.zeros((ZR, FP), f32)
    zeros_zrb = jnp.zeros((ZRB, FP), f32)
    ones_ch = jnp.ones((CH, FP), f32)
    sent = jnp.full((CAPT + 16,), -1, jnp.int32)
    xpad = jnp.concatenate([x, jnp.zeros((N, FP - 7), f32)], axis=1)

    stats = _sc_stats(srcp, dstp, zeros_zr, ones_ch)
    dinv, fill, xws0 = _tc_stats_finalize(stats, xpad)
    table0 = jnp.concatenate([xws0, jnp.zeros((AR - N, FP), f32)], axis=0)
    parts0 = _sc_aggregate(table0, srcp, dstp, zeros_zrb)
    W0p = jnp.concatenate([Wd0.T, jnp.zeros((FP - 7, H), f32)], axis=0)
    h1, s1 = _tc_conv0(parts0, xpad, dinv, fill, W0p, bd0[None, :],
                       p1[:, None])

    s1_row = jnp.concatenate([s1[:, 0], jnp.full((NJP - N,), -2.0, f32)])
    rank1_row, map1_row = _tc_rank(s1_row[None, :], s1, N, K1)
    rank1 = rank1_row[:, :N]
    map1p = jnp.concatenate(
        [map1_row[0, :N], jnp.full((AR - N,), -1, jnp.int32)])
    xp, perm1 = _tc_pool1(rank1, h1, s1)

    rl, rc, cl, cc = _sc_compact(srcp, dstp, map1p, sent)
    mr, mct = _sc_build(rl, rc, cl, cc, perm1[:, 0])
    b1 = _tc_b1(mr.reshape(K1P, NP), mct.reshape(K1P, NP))

    h2, s2, dinv1 = _tc_gcn1_down(b1, xp, Wd1.T, bd1[None, :], p2[:, None])
    rank2_row, _ = _tc_rank(s2[:, 0][None, :], s2, K1P, K2)
    xp2, b2 = _tc_level2(rank2_row, h2, s2, b1)
    hu = _tc_bottom_up(rank2_row, xp2, b2, h2, b1, dinv1, Wd2.T,
                       bd2[None, :], Wu0.T, bu0[None, :])

    Wu1p = jnp.concatenate([Wu1.T, jnp.zeros((H, FP - 7), f32)], axis=1)
    xwfs = _tc_unpool0(rank1.reshape(N, 1), h1, dinv, hu, Wu1p)
    tablef = jnp.concatenate([xwfs, jnp.zeros((AR - N, FP), f32)], axis=0)
    partsf = _sc_aggregate(tablef, srcp, dstp, zeros_zrb)
    out = _tc_final(partsf, xwfs, dinv, fill, bu1[None, :])
    return out
